# baseline (speedup 1.0000x reference)
import functools

import jax
import jax.numpy as jnp
from jax.experimental import pallas as pl
from jax.experimental.pallas import tpu as pltpu

N_ROWS = 5000
D_IN = 12544
D_HID = 1024
BR = 1000
BK = 896
NR = N_ROWS // BR
NK = D_IN // BK
C1 = 4
C4 = 12
CH = 16


def _boxhead_body(x_ref, w1_ref, b1_ref, w2_ref, b2_ref, wh_ref, bh_ref,
                  cls_ref, box_ref, acc_ref, w1b_ref, w2b_ref, whb_ref):
    k = pl.program_id(0)
    i = pl.program_id(1)
    rows = pl.ds(i * BR, BR)

    @pl.when(i == 0)
    def _cast_w1():
        w1b_ref[...] = w1_ref[...].astype(jnp.bfloat16)

    @pl.when(jnp.logical_and(k == 0, i == 0))
    def _cast_tail_weights():
        w2b_ref[...] = w2_ref[...].astype(jnp.bfloat16)
        whb_ref[...] = wh_ref[...].astype(jnp.bfloat16)

    xb = x_ref[...].astype(jnp.bfloat16)
    partial = jnp.dot(xb, w1b_ref[...], preferred_element_type=jnp.float32)

    @pl.when(k == 0)
    def _init():
        acc_ref[rows, :] = partial

    @pl.when(k > 0)
    def _accum():
        acc_ref[rows, :] += partial

    @pl.when(k == NK - 1)
    def _epilogue():
        h1 = jnp.maximum(acc_ref[rows, :] + b1_ref[...], 0.0)
        h2 = jnp.maximum(
            jnp.dot(h1.astype(jnp.bfloat16), w2b_ref[...],
                    preferred_element_type=jnp.float32)
            + b2_ref[...], 0.0)
        heads = (jnp.dot(h2.astype(jnp.bfloat16), whb_ref[...],
                         preferred_element_type=jnp.float32) + bh_ref[...])
        cls_ref[...] = heads[:, :C1]
        box_ref[...] = heads[:, C1:]


@functools.partial(jax.jit, static_argnames=())
def kernel(feature_vectors, W1, b1, W2, b2, W3, b3, W4, b4):
    WH = jnp.concatenate([W3, W4], axis=1)
    bh = jnp.concatenate([b3, b4]).reshape(1, CH)
    grid = (NK, NR)
    out = pl.pallas_call(
        _boxhead_body,
        grid=grid,
        in_specs=[
            pl.BlockSpec((BR, BK), lambda k, i: (i, k)),
            pl.BlockSpec((BK, D_HID), lambda k, i: (k, 0)),
            pl.BlockSpec((1, D_HID), lambda k, i: (0, 0)),
            pl.BlockSpec((D_HID, D_HID), lambda k, i: (0, 0)),
            pl.BlockSpec((1, D_HID), lambda k, i: (0, 0)),
            pl.BlockSpec((D_HID, CH), lambda k, i: (0, 0)),
            pl.BlockSpec((1, CH), lambda k, i: (0, 0)),
        ],
        out_specs=[
            pl.BlockSpec((BR, C1), lambda k, i: (i, 0)),
            pl.BlockSpec((BR, C4), lambda k, i: (i, 0)),
        ],
        out_shape=[
            jax.ShapeDtypeStruct((N_ROWS, C1), jnp.float32),
            jax.ShapeDtypeStruct((N_ROWS, C4), jnp.float32),
        ],
        scratch_shapes=[
            pltpu.VMEM((N_ROWS, D_HID), jnp.float32),
            pltpu.VMEM((BK, D_HID), jnp.bfloat16),
            pltpu.VMEM((D_HID, D_HID), jnp.bfloat16),
            pltpu.VMEM((D_HID, CH), jnp.bfloat16),
        ],
        compiler_params=pltpu.CompilerParams(
            dimension_semantics=("arbitrary", "arbitrary"),
        ),
    )(feature_vectors, W1, b1.reshape(1, -1), W2, b2.reshape(1, -1),
      WH, bh)
    return (out[0], out[1])


# speedup vs baseline: 1.1308x; 1.1308x over previous
import functools

import jax
import jax.numpy as jnp
from jax.experimental import pallas as pl
from jax.experimental.pallas import tpu as pltpu

N_ROWS = 5000
D_IN = 12544
D_HID = 1024
BR = 1000
BK = 1792
NR = N_ROWS // BR
NK = D_IN // BK
C1 = 4
C4 = 12
CH = 16


def _boxhead_body(x_ref, w1_ref, b1_ref, w2_ref, b2_ref, wh_ref, bh_ref,
                  cls_ref, box_ref, acc_ref):
    k = pl.program_id(1)

    xb = x_ref[...].astype(jnp.bfloat16)
    partial = jnp.dot(xb, w1_ref[...], preferred_element_type=jnp.float32)

    @pl.when(k == 0)
    def _init():
        acc_ref[...] = partial

    @pl.when(k > 0)
    def _accum():
        acc_ref[...] += partial

    @pl.when(k == NK - 1)
    def _epilogue():
        h1 = jnp.maximum(acc_ref[...] + b1_ref[...], 0.0)
        h2 = jnp.maximum(
            jnp.dot(h1.astype(jnp.bfloat16), w2_ref[...],
                    preferred_element_type=jnp.float32)
            + b2_ref[...], 0.0)
        heads = (jnp.dot(h2.astype(jnp.bfloat16), wh_ref[...],
                         preferred_element_type=jnp.float32) + bh_ref[...])
        cls_ref[...] = heads[:, :C1]
        box_ref[...] = heads[:, C1:]


@functools.partial(jax.jit, static_argnames=())
def kernel(feature_vectors, W1, b1, W2, b2, W3, b3, W4, b4):
    W1b = W1.astype(jnp.bfloat16)
    W2b = W2.astype(jnp.bfloat16)
    WHb = jnp.concatenate([W3, W4], axis=1).astype(jnp.bfloat16)
    bh = jnp.concatenate([b3, b4]).reshape(1, CH)
    grid = (NR, NK)
    out = pl.pallas_call(
        _boxhead_body,
        grid=grid,
        in_specs=[
            pl.BlockSpec((BR, BK), lambda i, k: (i, k)),
            pl.BlockSpec((BK, D_HID), lambda i, k: (k, 0)),
            pl.BlockSpec((1, D_HID), lambda i, k: (0, 0)),
            pl.BlockSpec((D_HID, D_HID), lambda i, k: (0, 0)),
            pl.BlockSpec((1, D_HID), lambda i, k: (0, 0)),
            pl.BlockSpec((D_HID, CH), lambda i, k: (0, 0)),
            pl.BlockSpec((1, CH), lambda i, k: (0, 0)),
        ],
        out_specs=[
            pl.BlockSpec((BR, C1), lambda i, k: (i, 0)),
            pl.BlockSpec((BR, C4), lambda i, k: (i, 0)),
        ],
        out_shape=[
            jax.ShapeDtypeStruct((N_ROWS, C1), jnp.float32),
            jax.ShapeDtypeStruct((N_ROWS, C4), jnp.float32),
        ],
        scratch_shapes=[
            pltpu.VMEM((BR, D_HID), jnp.float32),
        ],
        compiler_params=pltpu.CompilerParams(
            dimension_semantics=("arbitrary", "arbitrary"),
        ),
    )(feature_vectors, W1b, b1.reshape(1, -1), W2b, b2.reshape(1, -1),
      WHb, bh)
    return (out[0], out[1])


# speedup vs baseline: 1.1561x; 1.0224x over previous
import functools

import jax
import jax.numpy as jnp
from jax.experimental import pallas as pl
from jax.experimental.pallas import tpu as pltpu

N_ROWS = 5000
D_IN = 12544
D_HID = 1024
BR = 200
NR = N_ROWS // BR
C1 = 4
C4 = 12
CH = 16


def _boxhead_body(x_ref, w1_ref, b1_ref, w2_ref, b2_ref, wh_ref, bh_ref,
                  cls_ref, box_ref):
    xb = x_ref[...].astype(jnp.bfloat16)
    h1 = jnp.maximum(
        jnp.dot(xb, w1_ref[...], preferred_element_type=jnp.float32)
        + b1_ref[...], 0.0)
    h2 = jnp.maximum(
        jnp.dot(h1.astype(jnp.bfloat16), w2_ref[...],
                preferred_element_type=jnp.float32)
        + b2_ref[...], 0.0)
    heads = (jnp.dot(h2.astype(jnp.bfloat16), wh_ref[...],
                     preferred_element_type=jnp.float32) + bh_ref[...])
    cls_ref[...] = heads[:, :C1]
    box_ref[...] = heads[:, C1:]


@functools.partial(jax.jit, static_argnames=())
def kernel(feature_vectors, W1, b1, W2, b2, W3, b3, W4, b4):
    W1b = W1.astype(jnp.bfloat16)
    W2b = W2.astype(jnp.bfloat16)
    WHb = jnp.concatenate([W3, W4], axis=1).astype(jnp.bfloat16)
    bh = jnp.concatenate([b3, b4]).reshape(1, CH)
    out = pl.pallas_call(
        _boxhead_body,
        grid=(NR,),
        in_specs=[
            pl.BlockSpec((BR, D_IN), lambda i: (i, 0)),
            pl.BlockSpec((D_IN, D_HID), lambda i: (0, 0)),
            pl.BlockSpec((1, D_HID), lambda i: (0, 0)),
            pl.BlockSpec((D_HID, D_HID), lambda i: (0, 0)),
            pl.BlockSpec((1, D_HID), lambda i: (0, 0)),
            pl.BlockSpec((D_HID, CH), lambda i: (0, 0)),
            pl.BlockSpec((1, CH), lambda i: (0, 0)),
        ],
        out_specs=[
            pl.BlockSpec((BR, C1), lambda i: (i, 0)),
            pl.BlockSpec((BR, C4), lambda i: (i, 0)),
        ],
        out_shape=[
            jax.ShapeDtypeStruct((N_ROWS, C1), jnp.float32),
            jax.ShapeDtypeStruct((N_ROWS, C4), jnp.float32),
        ],
        compiler_params=pltpu.CompilerParams(
            dimension_semantics=("arbitrary",),
        ),
    )(feature_vectors, W1b, b1.reshape(1, -1), W2b, b2.reshape(1, -1),
      WHb, bh)
    return (out[0], out[1])


# speedup vs baseline: 1.1626x; 1.0056x over previous
import functools

import jax
import jax.numpy as jnp
from jax.experimental import pallas as pl
from jax.experimental.pallas import tpu as pltpu

N_ROWS = 5000
D_IN = 12544
D_HID = 1024
BR = 200
NR = N_ROWS // BR
WSLAB = 448
NW = D_IN // WSLAB
NSTEPS = NW + NR + 1
C1 = 4
C4 = 12
CH = 16


def _boxhead_body(x_ref, w1_ref, b1_ref, w2_ref, b2_ref, wh_ref, bh_ref,
                  cls_ref, box_ref, w1b_ref, h1_ref):
    j = pl.program_id(0)

    @pl.when(j < NW)
    def _cast_w1():
        w1b_ref[pl.ds(j * WSLAB, WSLAB), :] = w1_ref[...].astype(jnp.bfloat16)

    @pl.when(jnp.logical_and(j >= NW, j < NW + NR))
    def _layer1():
        xb = x_ref[...].astype(jnp.bfloat16)
        pre = jnp.dot(xb, w1b_ref[...], preferred_element_type=jnp.float32)
        h1_ref[(j - NW) % 2] = jnp.maximum(
            pre + b1_ref[...], 0.0).astype(jnp.bfloat16)

    @pl.when(j >= NW + 1)
    def _epilogue():
        h2 = jnp.maximum(
            jnp.dot(h1_ref[(j - NW - 1) % 2], w2_ref[...],
                    preferred_element_type=jnp.float32)
            + b2_ref[...], 0.0)
        heads = (jnp.dot(h2.astype(jnp.bfloat16), wh_ref[...],
                         preferred_element_type=jnp.float32) + bh_ref[...])
        cls_ref[...] = heads[:, :C1]
        box_ref[...] = heads[:, C1:]


def _clamp(lo, v, hi):
    return jnp.minimum(jnp.maximum(v, lo), hi)


@functools.partial(jax.jit, static_argnames=())
def kernel(feature_vectors, W1, b1, W2, b2, W3, b3, W4, b4):
    W2b = W2.astype(jnp.bfloat16)
    WHb = jnp.concatenate([W3, W4], axis=1).astype(jnp.bfloat16)
    bh = jnp.concatenate([b3, b4]).reshape(1, CH)
    out = pl.pallas_call(
        _boxhead_body,
        grid=(NSTEPS,),
        in_specs=[
            pl.BlockSpec((BR, D_IN),
                         lambda j: (_clamp(0, j - NW, NR - 1), 0)),
            pl.BlockSpec((WSLAB, D_HID),
                         lambda j: (_clamp(0, j, NW - 1), 0)),
            pl.BlockSpec((1, D_HID), lambda j: (0, 0)),
            pl.BlockSpec((D_HID, D_HID), lambda j: (0, 0)),
            pl.BlockSpec((1, D_HID), lambda j: (0, 0)),
            pl.BlockSpec((D_HID, CH), lambda j: (0, 0)),
            pl.BlockSpec((1, CH), lambda j: (0, 0)),
        ],
        out_specs=[
            pl.BlockSpec((BR, C1), lambda j: (_clamp(0, j - NW - 1, NR - 1), 0)),
            pl.BlockSpec((BR, C4), lambda j: (_clamp(0, j - NW - 1, NR - 1), 0)),
        ],
        out_shape=[
            jax.ShapeDtypeStruct((N_ROWS, C1), jnp.float32),
            jax.ShapeDtypeStruct((N_ROWS, C4), jnp.float32),
        ],
        scratch_shapes=[
            pltpu.VMEM((D_IN, D_HID), jnp.bfloat16),
            pltpu.VMEM((2, BR, D_HID), jnp.bfloat16),
        ],
        compiler_params=pltpu.CompilerParams(
            dimension_semantics=("arbitrary",),
        ),
    )(feature_vectors, W1, b1.reshape(1, -1), W2b, b2.reshape(1, -1),
      WHb, bh)
    return (out[0], out[1])
